# Initial kernel scaffold; baseline (speedup 1.0000x reference)
import jax
import jax.numpy as jnp
from jax.experimental import pallas as pl


def kernel(indices, weight):
    raise NotImplementedError("write your pallas kernel here")


# speedup vs baseline: 2.9702x; 2.9702x over previous
import functools

import jax
import jax.numpy as jnp
from jax import lax
from jax.experimental import pallas as pl
from jax.experimental.pallas import tpu as pltpu
from jax.experimental.pallas import tpu_sc as plsc

_CHUNK = 128


@functools.lru_cache(maxsize=None)
def _make_gather(num_rows, dim, table_rows):
    info = plsc.get_sparse_core_info()
    nc, ns = info.num_cores, info.num_subcores
    nw = nc * ns
    assert num_rows % (nw * _CHUNK) == 0
    cpw = num_rows // (nw * _CHUNK)

    mesh = plsc.VectorSubcoreMesh(core_axis_name="c", subcore_axis_name="s")

    @functools.partial(
        pl.kernel,
        mesh=mesh,
        out_type=jax.ShapeDtypeStruct((num_rows, dim), jnp.float32),
        scratch_types=[
            pltpu.VMEM((cpw * _CHUNK,), jnp.int32),
            pltpu.VMEM((_CHUNK, dim), jnp.float32),
            pltpu.SemaphoreType.DMA,
        ],
    )
    def gather_k(table_hbm, idx_hbm, out_hbm, idx_v, rows_v, sem):
        wid = lax.axis_index("s") * nc + lax.axis_index("c")
        base = wid * cpw
        pltpu.sync_copy(idx_hbm.at[pl.ds(base * _CHUNK, cpw * _CHUNK)], idx_v)

        def step(t, carry):
            pltpu.async_copy(
                table_hbm.at[idx_v.at[pl.ds(t * _CHUNK, _CHUNK)]],
                rows_v, sem).wait()
            pltpu.sync_copy(
                rows_v, out_hbm.at[pl.ds((base + t) * _CHUNK, _CHUNK)])
            return carry

        lax.fori_loop(0, cpw, step, 0)

    return gather_k


def kernel(indices, weight):
    table_rows, dim = weight.shape
    idx = indices.reshape(-1).astype(jnp.int32)
    num_rows = idx.shape[0]
    out = _make_gather(num_rows, dim, table_rows)(weight, idx)
    return out.reshape(indices.shape + (dim,))


# speedup vs baseline: 3.1241x; 1.0518x over previous
import functools

import jax
import jax.numpy as jnp
from jax import lax
from jax.experimental import pallas as pl
from jax.experimental.pallas import tpu as pltpu
from jax.experimental.pallas import tpu_sc as plsc

_CHUNK = 128


@functools.lru_cache(maxsize=None)
def _make_gather(num_rows, dim, table_rows):
    info = plsc.get_sparse_core_info()
    nc, ns = info.num_cores, info.num_subcores
    nw = nc * ns
    assert num_rows % (nw * _CHUNK) == 0
    cpw = num_rows // (nw * _CHUNK)

    mesh = plsc.VectorSubcoreMesh(core_axis_name="c", subcore_axis_name="s")

    @functools.partial(
        pl.kernel,
        mesh=mesh,
        out_type=jax.ShapeDtypeStruct((num_rows, dim), jnp.float32),
        scratch_types=[
            pltpu.VMEM((cpw * _CHUNK,), jnp.int32),
            pltpu.VMEM((_CHUNK, dim), jnp.float32),
            pltpu.VMEM((_CHUNK, dim), jnp.float32),
            pltpu.SemaphoreType.DMA,
            pltpu.SemaphoreType.DMA,
        ],
    )
    def gather_k(table_hbm, idx_hbm, out_hbm, idx_v, rows0_v, rows1_v,
                 sem0, sem1):
        wid = lax.axis_index("s") * nc + lax.axis_index("c")
        base = wid * cpw
        pltpu.sync_copy(idx_hbm.at[pl.ds(base * _CHUNK, cpw * _CHUNK)], idx_v)

        bufs = (rows0_v, rows1_v)
        sems = (sem0, sem1)

        def start_gather(t, b):
            return pltpu.async_copy(
                table_hbm.at[idx_v.at[pl.ds(t * _CHUNK, _CHUNK)]],
                bufs[b], sems[b])

        start_gather(0, 0)

        def outer(tt, carry):
            for b in range(2):
                t = tt * 2 + b
                pltpu.make_async_copy(
                    table_hbm.at[idx_v.at[pl.ds(t * _CHUNK, _CHUNK)]],
                    bufs[b], sems[b]).wait()

                @pl.when(t + 1 < cpw)
                def _():
                    start_gather(t + 1, 1 - b)

                pltpu.sync_copy(
                    bufs[b], out_hbm.at[pl.ds((base + t) * _CHUNK, _CHUNK)])
            return carry

        lax.fori_loop(0, cpw // 2, outer, 0)

    return gather_k


def kernel(indices, weight):
    table_rows, dim = weight.shape
    idx = indices.reshape(-1).astype(jnp.int32)
    num_rows = idx.shape[0]
    out = _make_gather(num_rows, dim, table_rows)(weight, idx)
    return out.reshape(indices.shape + (dim,))


# speedup vs baseline: 4.3427x; 1.3901x over previous
import functools

import jax
import jax.numpy as jnp
from jax import lax
from jax.experimental import pallas as pl
from jax.experimental.pallas import tpu as pltpu
from jax.experimental.pallas import tpu_sc as plsc

_PAD = 56
_CPB = 2


@functools.lru_cache(maxsize=None)
def _make_gather(batch, seq, dim, table_rows):
    info = plsc.get_sparse_core_info()
    nc, ns = info.num_cores, info.num_subcores
    nw = nc * ns
    assert batch % (nw * _CPB) == 0
    rpw = batch // nw
    cpw = rpw // _CPB
    nidx = _CPB * _PAD

    mesh = plsc.VectorSubcoreMesh(core_axis_name="c", subcore_axis_name="s")

    @functools.partial(
        pl.kernel,
        mesh=mesh,
        out_type=jax.ShapeDtypeStruct((batch, seq, dim), jnp.float32),
        scratch_types=[
            pltpu.VMEM((rpw * _PAD,), jnp.int32),
            pltpu.VMEM((nidx, dim), jnp.float32),
            pltpu.VMEM((nidx, dim), jnp.float32),
            pltpu.SemaphoreType.DMA,
            pltpu.SemaphoreType.DMA,
        ],
        compiler_params=pltpu.CompilerParams(use_tc_tiling_on_sc=True),
    )
    def gather_k(table_hbm, idx_hbm, out_hbm, idx_v, rows0_v, rows1_v,
                 sem0, sem1):
        wid = lax.axis_index("s") * nc + lax.axis_index("c")
        row0 = wid * rpw
        pltpu.sync_copy(idx_hbm.at[pl.ds(row0 * _PAD, rpw * _PAD)], idx_v)

        bufs = (rows0_v, rows1_v)
        sems = (sem0, sem1)

        def start_gather(t, b):
            return pltpu.async_copy(
                table_hbm.at[idx_v.at[pl.ds(t * nidx, nidx)]],
                bufs[b], sems[b])

        start_gather(0, 0)

        def outer(tt, carry):
            for b in range(2):
                t = tt * 2 + b
                pltpu.make_async_copy(
                    table_hbm.at[idx_v.at[pl.ds(t * nidx, nidx)]],
                    bufs[b], sems[b]).wait()

                @pl.when(t + 1 < cpw)
                def _():
                    start_gather(t + 1, 1 - b)

                b0 = row0 + t * _CPB
                pltpu.sync_copy(bufs[b].at[pl.ds(0, seq)], out_hbm.at[b0])
                pltpu.sync_copy(bufs[b].at[pl.ds(_PAD, seq)],
                                out_hbm.at[b0 + 1])
            return carry

        lax.fori_loop(0, cpw // 2, outer, 0)

    return gather_k


def kernel(indices, weight):
    table_rows, dim = weight.shape
    batch, seq = indices.shape
    idx = indices.astype(jnp.int32)
    idxp = jnp.pad(idx, ((0, 0), (0, _PAD - seq)), mode="edge")
    return _make_gather(batch, seq, dim, table_rows)(
        weight, idxp.reshape(-1))


# speedup vs baseline: 5.7010x; 1.3128x over previous
import functools

import jax
import jax.numpy as jnp
from jax import lax
from jax.experimental import pallas as pl
from jax.experimental.pallas import tpu as pltpu
from jax.experimental.pallas import tpu_sc as plsc

_CPB = 4
_SPLIT = (104, 96)


@functools.lru_cache(maxsize=None)
def _make_gather(batch, seq, dim, table_rows):
    info = plsc.get_sparse_core_info()
    nc, ns = info.num_cores, info.num_subcores
    nw = nc * ns
    assert batch % (nw * _CPB) == 0
    rpw = batch // nw
    cpw = rpw // _CPB
    nidx = _CPB * seq
    assert sum(_SPLIT) == nidx

    mesh = plsc.VectorSubcoreMesh(core_axis_name="c", subcore_axis_name="s")

    @functools.partial(
        pl.kernel,
        mesh=mesh,
        out_type=jax.ShapeDtypeStruct((batch, seq, dim), jnp.float32),
        scratch_types=[
            pltpu.VMEM((rpw * seq,), jnp.int32),
            pltpu.VMEM((nidx, dim), jnp.float32),
            pltpu.VMEM((nidx, dim), jnp.float32),
            pltpu.SemaphoreType.DMA,
            pltpu.SemaphoreType.DMA,
        ],
        compiler_params=pltpu.CompilerParams(use_tc_tiling_on_sc=True),
    )
    def gather_k(table_hbm, idx_hbm, out_hbm, idx_v, rows0_v, rows1_v,
                 sem0, sem1):
        wid = lax.axis_index("s") * nc + lax.axis_index("c")
        row0 = wid * rpw
        pltpu.sync_copy(idx_hbm.at[pl.ds(row0 * seq, rpw * seq)], idx_v)

        bufs = (rows0_v, rows1_v)
        sems = (sem0, sem1)

        def chunk_copies(t, b):
            copies = []
            off = 0
            for n in _SPLIT:
                copies.append(pltpu.make_async_copy(
                    table_hbm.at[idx_v.at[pl.ds(t * nidx + off, n)]],
                    bufs[b].at[pl.ds(off, n)], sems[b]))
                off += n
            return copies

        def start_chunk(t, b):
            for c in chunk_copies(t, b):
                c.start()

        start_chunk(0, 0)

        def outer(tt, carry):
            for b in range(2):
                t = tt * 2 + b
                for c in chunk_copies(t, b):
                    c.wait()

                @pl.when(t + 1 < cpw)
                def _():
                    start_chunk(t + 1, 1 - b)

                b0 = row0 + t * _CPB
                for r in range(_CPB):
                    pltpu.sync_copy(bufs[b].at[pl.ds(r * seq, seq)],
                                    out_hbm.at[b0 + r])
            return carry

        lax.fori_loop(0, cpw // 2, outer, 0)

    return gather_k


def kernel(indices, weight):
    table_rows, dim = weight.shape
    batch, seq = indices.shape
    idx = indices.astype(jnp.int32).reshape(-1)
    return _make_gather(batch, seq, dim, table_rows)(weight, idx)


# speedup vs baseline: 8.6554x; 1.5182x over previous
import functools

import jax
import jax.numpy as jnp
from jax import lax
from jax.experimental import pallas as pl
from jax.experimental.pallas import tpu as pltpu
from jax.experimental.pallas import tpu_sc as plsc

_CHUNK = 128


@functools.lru_cache(maxsize=None)
def _make_gather(num_rows, dim, table_rows):
    info = plsc.get_sparse_core_info()
    nc, ns = info.num_cores, info.num_subcores
    nw = nc * ns
    assert num_rows % (nw * _CHUNK) == 0
    cpw = num_rows // (nw * _CHUNK)

    mesh = plsc.VectorSubcoreMesh(core_axis_name="c", subcore_axis_name="s")

    @functools.partial(
        pl.kernel,
        mesh=mesh,
        out_type=jax.ShapeDtypeStruct((num_rows, dim), jnp.float32),
        scratch_types=[
            pltpu.VMEM((cpw * _CHUNK,), jnp.int32),
            pltpu.VMEM((_CHUNK, dim), jnp.float32),
            pltpu.VMEM((_CHUNK, dim), jnp.float32),
            pltpu.SemaphoreType.DMA,
            pltpu.SemaphoreType.DMA,
        ],
        compiler_params=pltpu.CompilerParams(use_tc_tiling_on_sc=True),
    )
    def gather_k(table_hbm, idx_hbm, out_hbm, idx_v, rows0_v, rows1_v,
                 sem0, sem1):
        wid = lax.axis_index("s") * nc + lax.axis_index("c")
        base = wid * cpw
        pltpu.sync_copy(idx_hbm.at[pl.ds(base * _CHUNK, cpw * _CHUNK)], idx_v)

        bufs = (rows0_v, rows1_v)
        sems = (sem0, sem1)

        def gather_copy(t, b):
            return pltpu.make_async_copy(
                table_hbm.at[idx_v.at[pl.ds(t * _CHUNK, _CHUNK)]],
                bufs[b], sems[b])

        gather_copy(0, 0).start()

        def outer(tt, carry):
            for b in range(2):
                t = tt * 2 + b
                gather_copy(t, b).wait()

                @pl.when(t + 1 < cpw)
                def _():
                    gather_copy(t + 1, 1 - b).start()

                pltpu.sync_copy(
                    bufs[b], out_hbm.at[pl.ds((base + t) * _CHUNK, _CHUNK)])
            return carry

        lax.fori_loop(0, cpw // 2, outer, 0)

    return gather_k


def kernel(indices, weight):
    table_rows, dim = weight.shape
    batch, seq = indices.shape
    idx = indices.T.reshape(-1).astype(jnp.int32)
    out = _make_gather(batch * seq, dim, table_rows)(weight, idx)
    return out.reshape(seq, batch, dim).transpose(1, 0, 2)


# speedup vs baseline: 10.1928x; 1.1776x over previous
import functools

import jax
import jax.numpy as jnp
from jax import lax
from jax.experimental import pallas as pl
from jax.experimental.pallas import tpu as pltpu
from jax.experimental.pallas import tpu_sc as plsc

_GATH = 128
_CHUNK = 256


@functools.lru_cache(maxsize=None)
def _make_gather(num_rows, dim, table_rows):
    info = plsc.get_sparse_core_info()
    nc, ns = info.num_cores, info.num_subcores
    nw = nc * ns
    assert num_rows % (nw * _CHUNK) == 0
    cpw = num_rows // (nw * _CHUNK)

    mesh = plsc.VectorSubcoreMesh(core_axis_name="c", subcore_axis_name="s")

    @functools.partial(
        pl.kernel,
        mesh=mesh,
        out_type=jax.ShapeDtypeStruct((num_rows, dim), jnp.float32),
        scratch_types=[
            pltpu.VMEM((cpw * _CHUNK,), jnp.int32),
            pltpu.VMEM((_CHUNK, dim), jnp.float32),
            pltpu.VMEM((_CHUNK, dim), jnp.float32),
            pltpu.SemaphoreType.DMA,
            pltpu.SemaphoreType.DMA,
        ],
        compiler_params=pltpu.CompilerParams(use_tc_tiling_on_sc=True),
    )
    def gather_k(table_hbm, idx_hbm, out_hbm, idx_v, rows0_v, rows1_v,
                 sem0, sem1):
        wid = lax.axis_index("s") * nc + lax.axis_index("c")
        base = wid * cpw
        pltpu.sync_copy(idx_hbm.at[pl.ds(base * _CHUNK, cpw * _CHUNK)], idx_v)

        bufs = (rows0_v, rows1_v)
        sems = (sem0, sem1)

        def gather_copies(t, b):
            return [
                pltpu.make_async_copy(
                    table_hbm.at[idx_v.at[pl.ds(t * _CHUNK + off, _GATH)]],
                    bufs[b].at[pl.ds(off, _GATH)], sems[b])
                for off in range(0, _CHUNK, _GATH)
            ]

        def start_chunk(t, b):
            for c in gather_copies(t, b):
                c.start()

        start_chunk(0, 0)

        def outer(tt, carry):
            for b in range(2):
                t = tt * 2 + b
                for c in gather_copies(t, b):
                    c.wait()

                @pl.when(t + 1 < cpw)
                def _():
                    start_chunk(t + 1, 1 - b)

                pltpu.sync_copy(
                    bufs[b], out_hbm.at[pl.ds((base + t) * _CHUNK, _CHUNK)])
            return carry

        lax.fori_loop(0, cpw // 2, outer, 0)

        if cpw % 2:
            t = cpw - 1
            b = t % 2
            for c in gather_copies(t, b):
                c.wait()
            pltpu.sync_copy(
                bufs[b], out_hbm.at[pl.ds((base + t) * _CHUNK, _CHUNK)])

    return gather_k


def kernel(indices, weight):
    table_rows, dim = weight.shape
    batch, seq = indices.shape
    idx = indices.T.reshape(-1).astype(jnp.int32)
    out = _make_gather(batch * seq, dim, table_rows)(weight, idx)
    return out.reshape(seq, batch, dim).transpose(1, 0, 2)


# speedup vs baseline: 10.4875x; 1.0289x over previous
import functools

import jax
import jax.numpy as jnp
from jax import lax
from jax.experimental import pallas as pl
from jax.experimental.pallas import tpu as pltpu
from jax.experimental.pallas import tpu_sc as plsc

_GATH = 128
_CHUNK = 256
_NBUF = 3


@functools.lru_cache(maxsize=None)
def _make_gather(num_rows, dim, table_rows):
    info = plsc.get_sparse_core_info()
    nc, ns = info.num_cores, info.num_subcores
    nw = nc * ns
    assert num_rows % (nw * _CHUNK) == 0
    cpw = num_rows // (nw * _CHUNK)

    mesh = plsc.VectorSubcoreMesh(core_axis_name="c", subcore_axis_name="s")

    @functools.partial(
        pl.kernel,
        mesh=mesh,
        out_type=jax.ShapeDtypeStruct((num_rows, dim), jnp.float32),
        scratch_types=(
            [pltpu.VMEM((cpw * _CHUNK,), jnp.int32)]
            + [pltpu.VMEM((_CHUNK, dim), jnp.float32)] * _NBUF
            + [pltpu.SemaphoreType.DMA] * _NBUF
        ),
        compiler_params=pltpu.CompilerParams(use_tc_tiling_on_sc=True),
    )
    def gather_k(table_hbm, idx_hbm, out_hbm, idx_v, *bufs_and_sems):
        bufs = bufs_and_sems[:_NBUF]
        sems = bufs_and_sems[_NBUF:]
        wid = lax.axis_index("s") * nc + lax.axis_index("c")
        base = wid * cpw
        pltpu.sync_copy(idx_hbm.at[pl.ds(base * _CHUNK, cpw * _CHUNK)], idx_v)

        def gather_copies(t, b):
            return [
                pltpu.make_async_copy(
                    table_hbm.at[idx_v.at[pl.ds(t * _CHUNK + off, _GATH)]],
                    bufs[b].at[pl.ds(off, _GATH)], sems[b])
                for off in range(0, _CHUNK, _GATH)
            ]

        def start_chunk(t, b):
            for c in gather_copies(t, b):
                c.start()

        def finish_chunk(t, b):
            for c in gather_copies(t, b):
                c.wait()
            pltpu.sync_copy(
                bufs[b], out_hbm.at[pl.ds((base + t) * _CHUNK, _CHUNK)])

        for p in range(_NBUF - 1):
            start_chunk(p, p)

        def outer(tt, carry):
            for b in range(_NBUF):
                t = tt * _NBUF + b
                for c in gather_copies(t, b):
                    c.wait()

                @pl.when(t + _NBUF - 1 < cpw)
                def _():
                    start_chunk(t + _NBUF - 1, (t + _NBUF - 1) % _NBUF)

                pltpu.sync_copy(
                    bufs[b], out_hbm.at[pl.ds((base + t) * _CHUNK, _CHUNK)])
            return carry

        lax.fori_loop(0, cpw // _NBUF, outer, 0)

        for t in range(cpw - cpw % _NBUF, cpw):
            finish_chunk(t, t % _NBUF)

    return gather_k


def kernel(indices, weight):
    table_rows, dim = weight.shape
    batch, seq = indices.shape
    idx = indices.T.reshape(-1).astype(jnp.int32)
    out = _make_gather(batch * seq, dim, table_rows)(weight, idx)
    return out.reshape(seq, batch, dim).transpose(1, 0, 2)


# speedup vs baseline: 10.5038x; 1.0016x over previous
import functools

import jax
import jax.numpy as jnp
from jax import lax
from jax.experimental import pallas as pl
from jax.experimental.pallas import tpu as pltpu
from jax.experimental.pallas import tpu_sc as plsc

_GATH = 128
_CHUNK = 256
_NBUF = 3


@functools.lru_cache(maxsize=None)
def _make_gather(num_rows, dim, table_rows):
    info = plsc.get_sparse_core_info()
    nc, ns = info.num_cores, info.num_subcores
    nw = nc * ns
    assert num_rows % (nw * _CHUNK) == 0
    cpw = num_rows // (nw * _CHUNK)

    mesh = plsc.VectorSubcoreMesh(core_axis_name="c", subcore_axis_name="s")

    @functools.partial(
        pl.kernel,
        mesh=mesh,
        out_type=jax.ShapeDtypeStruct((num_rows, dim), jnp.float32),
        scratch_types=(
            [pltpu.VMEM((cpw * _CHUNK,), jnp.int32)]
            + [pltpu.VMEM((_CHUNK, dim), jnp.float32)] * _NBUF
            + [pltpu.SemaphoreType.DMA] * _NBUF
        ),
        compiler_params=pltpu.CompilerParams(use_tc_tiling_on_sc=True),
    )
    def gather_k(table_hbm, idx_hbm, out_hbm, idx_v, *bufs_and_sems):
        bufs = bufs_and_sems[:_NBUF]
        sems = bufs_and_sems[_NBUF:]
        wid = lax.axis_index("s") * nc + lax.axis_index("c")
        base = wid * cpw
        pltpu.sync_copy(idx_hbm.at[pl.ds(base * _CHUNK, cpw * _CHUNK)], idx_v)

        def gather_copies(t, b):
            return [
                pltpu.make_async_copy(
                    table_hbm.at[idx_v.at[pl.ds(t * _CHUNK + off, _GATH)]],
                    bufs[b].at[pl.ds(off, _GATH)], sems[b])
                for off in range(0, _CHUNK, _GATH)
            ]

        def start_chunk(t, b):
            for c in gather_copies(t, b):
                c.start()

        def finish_chunk(t, b):
            for c in gather_copies(t, b):
                c.wait()
            pltpu.sync_copy(
                bufs[b], out_hbm.at[pl.ds((base + t) * _CHUNK, _CHUNK)])

        for p in range(_NBUF - 1):
            start_chunk(p, p)

        def outer(tt, carry):
            for b in range(_NBUF):
                t = tt * _NBUF + b
                for c in gather_copies(t, b):
                    c.wait()

                @pl.when(t + _NBUF - 1 < cpw)
                def _():
                    start_chunk(t + _NBUF - 1, (b + _NBUF - 1) % _NBUF)

                pltpu.sync_copy(
                    bufs[b], out_hbm.at[pl.ds((base + t) * _CHUNK, _CHUNK)])
            return carry

        lax.fori_loop(0, cpw // _NBUF, outer, 0)

        for t in range(cpw - cpw % _NBUF, cpw):
            finish_chunk(t, t % _NBUF)

    return gather_k


def kernel(indices, weight):
    table_rows, dim = weight.shape
    batch, seq = indices.shape
    idx = indices.T.reshape(-1).astype(jnp.int32)
    out = _make_gather(batch * seq, dim, table_rows)(weight, idx)
    return out.reshape(seq, batch, dim).transpose(1, 0, 2)
